# Initial kernel scaffold; baseline (speedup 1.0000x reference)
import jax
import jax.numpy as jnp
from jax.experimental import pallas as pl


def kernel(x):
    raise NotImplementedError("write your pallas kernel here")


# speedup vs baseline: 4.9402x; 4.9402x over previous
import functools

import jax
import jax.numpy as jnp
from jax import lax
from jax.experimental import pallas as pl
from jax.experimental.pallas import tpu as pltpu
from jax.experimental.pallas import tpu_sc as plsc

_GRID = 7
_NCELL = _GRID * _GRID
_NPAD = 64
_NCH = 30
_NCLS = 20
_STRIDE = 64.0
_CONF_T = 0.5
_IOU_T = 0.5
_L = 16
_NCHUNK = _NPAD // _L

_GDN = lax.GatherDimensionNumbers(
    offset_dims=(), collapsed_slice_dims=(0,), start_index_map=(0,))


def _sig(v):
    return 1.0 / (1.0 + jnp.exp(-v))


def _perm(v, idx):
    return lax.gather(v, idx.reshape(_L, 1), _GDN, (1,),
                      mode=lax.GatherScatterMode.PROMISE_IN_BOUNDS)


def _xlane(v, op, lane):
    for s in (1, 2, 4, 8):
        v = op(v, _perm(v, lane ^ s))
    return v


def _yolo_body(x_hbm, out_hbm, xv, x1r, y1r, x2r, y2r, arear, cxr, cyr, wr,
               hr, confr, clsr, keepr, crr, outv):
    @pl.when((lax.axis_index("c") == 0) & (lax.axis_index("s") == 0))
    def _():
        pltpu.sync_copy(x_hbm, xv)

        lane = lax.iota(jnp.int32, _L)
        zeros = jnp.zeros((_L,), jnp.float32)
        for j in range(_NCHUNK):
            sl = pl.ds(_L * j, _L)
            idxv = lane + _L * j
            inb = idxv < _NCELL
            c0 = _sig(xv[_NCLS, sl])
            c1 = _sig(xv[_NCLS + 5, sl])
            sel = c0 >= c1
            conf = jnp.where(sel, c0, c1)
            bx = _sig(jnp.where(sel, xv[_NCLS + 1, sl], xv[_NCLS + 6, sl]))
            by = _sig(jnp.where(sel, xv[_NCLS + 2, sl], xv[_NCLS + 7, sl]))
            bw = _sig(jnp.where(sel, xv[_NCLS + 3, sl], xv[_NCLS + 8, sl]))
            bh = _sig(jnp.where(sel, xv[_NCLS + 4, sl], xv[_NCLS + 9, sl]))
            gx = (idxv % _GRID).astype(jnp.float32)
            gy = (idxv // _GRID).astype(jnp.float32)
            cx = (bx + gx) * _STRIDE
            cy = (by + gy) * _STRIDE
            w = (bw * float(_GRID)) * _STRIDE
            h = (bh * float(_GRID)) * _STRIDE
            mv = _sig(xv[0, sl])
            cl = zeros
            for c in range(1, _NCLS):
                v = _sig(xv[c, sl])
                cl = jnp.where(v > mv, float(c), cl)
                mv = jnp.maximum(mv, v)
            x1 = cx - w / 2.0
            y1 = cy - h / 2.0
            x2 = cx + w / 2.0
            y2 = cy + h / 2.0
            x1r[sl] = x1
            y1r[sl] = y1
            x2r[sl] = x2
            y2r[sl] = y2
            arear[sl] = (x2 - x1) * (y2 - y1)
            cxr[sl] = cx
            cyr[sl] = cy
            wr[sl] = w
            hr[sl] = h
            confr[sl] = conf
            clsr[sl] = cl
            keepr[sl] = zeros
            crr[sl] = jnp.where(inb, conf, 0.0)

        ones = jnp.ones((_L,), jnp.float32)
        lane0 = lane == 0

        def nms_step(i, carry):
            mxv = crr[pl.ds(0, _L)]
            for j in range(1, _NCHUNK):
                mxv = jnp.maximum(mxv, crr[pl.ds(_L * j, _L)])
            mxv = _xlane(mxv, jnp.maximum, lane)
            active = mxv > _CONF_T
            cm = jnp.int32(_NPAD)
            candv = jnp.full((_L,), _NPAD, jnp.int32)
            for j in range(_NCHUNK):
                idxv = lane + _L * j
                hit = jnp.where(crr[pl.ds(_L * j, _L)] == mxv, idxv, _NPAD - 1)
                candv = jnp.minimum(candv, hit)
            candv = _xlane(candv, jnp.minimum, lane)
            plsc.store_scatter(keepr, [candv], ones, mask=lane0 & active)
            bx1 = plsc.load_gather(x1r, [candv])
            by1 = plsc.load_gather(y1r, [candv])
            bx2 = plsc.load_gather(x2r, [candv])
            by2 = plsc.load_gather(y2r, [candv])
            ba = plsc.load_gather(arear, [candv])
            for j in range(_NCHUNK):
                sl = pl.ds(_L * j, _L)
                idxv = lane + _L * j
                ix1 = jnp.maximum(x1r[sl], bx1)
                iy1 = jnp.maximum(y1r[sl], by1)
                ix2 = jnp.minimum(x2r[sl], bx2)
                iy2 = jnp.minimum(y2r[sl], by2)
                inter = jnp.maximum(ix2 - ix1, 0.0) * jnp.maximum(iy2 - iy1, 0.0)
                iou = inter / (arear[sl] + ba - inter + 1e-9)
                gone = ((iou > _IOU_T) | (idxv == candv)) & active
                crr[sl] = jnp.where(gone, 0.0, crr[sl])
            return carry

        lax.fori_loop(0, _NCELL, nms_step, jnp.int32(0))

        for j in range(_NCHUNK):
            sl = pl.ds(_L * j, _L)
            idxv = lane + _L * j
            inb = idxv < _NCELL
            kp = keepr[sl]
            for col, ref in enumerate((cxr, cyr, wr, hr, confr, clsr)):
                colv = jnp.full((_L,), col, jnp.int32)
                plsc.store_scatter(outv, [idxv, colv], ref[sl] * kp, mask=inb)

        pltpu.sync_copy(outv, out_hbm)


_vmem64 = lambda: pltpu.VMEM((_NPAD,), jnp.float32)

_yolo_sc = functools.partial(
    pl.kernel,
    out_type=jax.ShapeDtypeStruct((_NCELL, 6), jnp.float32),
    mesh=plsc.VectorSubcoreMesh(core_axis_name="c", subcore_axis_name="s"),
    compiler_params=pltpu.CompilerParams(needs_layout_passes=False),
    scratch_types=[
        pltpu.VMEM((32, _NPAD), jnp.float32),
        _vmem64(), _vmem64(), _vmem64(), _vmem64(), _vmem64(),
        _vmem64(), _vmem64(), _vmem64(), _vmem64(),
        _vmem64(), _vmem64(), _vmem64(), _vmem64(),
        pltpu.VMEM((_NCELL, 6), jnp.float32),
    ],
)(_yolo_body)


@jax.jit
def kernel(x):
    xt = jnp.transpose(x.reshape(_NCELL, _NCH))
    xt = jnp.pad(xt, ((0, 32 - _NCH), (0, _NPAD - _NCELL)))
    return _yolo_sc(xt)


# speedup vs baseline: 5.0175x; 1.0157x over previous
import functools

import jax
import jax.numpy as jnp
from jax import lax
from jax.experimental import pallas as pl
from jax.experimental.pallas import tpu as pltpu
from jax.experimental.pallas import tpu_sc as plsc

_GRID = 7
_NCELL = _GRID * _GRID
_NCH = 30
_NCLS = 20
_STRIDE = 64.0
_CONF_T = 0.5
_IOU_T = 0.5
_L = 16
_NCHUNK = 4

_GDN = lax.GatherDimensionNumbers(
    offset_dims=(), collapsed_slice_dims=(0,), start_index_map=(0,))


def _sig(v):
    return 1.0 / (1.0 + jnp.exp(-v))


def _perm(v, idx):
    return lax.gather(v, idx.reshape(_L, 1), _GDN, (1,),
                      mode=lax.GatherScatterMode.PROMISE_IN_BOUNDS)


def _xlane(v, op, lane):
    for s in (1, 2, 4, 8):
        v = op(v, _perm(v, lane ^ s))
    return v


def _yolo_body(x_hbm, out_hbm, xv, x1r, y1r, x2r, y2r, arear, cxr, cyr, wr,
               hr, confr, clsr, keepr, outv):
    @pl.when((lax.axis_index("c") == 0) & (lax.axis_index("s") == 0))
    def _():
        pltpu.sync_copy(x_hbm, xv)

        lane = lax.iota(jnp.int32, _L)
        zeros = jnp.zeros((_L,), jnp.float32)
        zeroi = jnp.zeros((_L,), jnp.int32)
        x1v, y1v, x2v, y2v, areav, crv = [], [], [], [], [], []
        for j in range(_NCHUNK):
            sl = pl.ds(_L * j, _L)
            idxv = lane + _L * j
            inb = idxv < _NCELL
            base = idxv * _NCH
            if j == _NCHUNK - 1:
                ch = lambda c: plsc.load_gather(xv, [zeroi, base + c], mask=inb)
            else:
                ch = lambda c: plsc.load_gather(xv, [zeroi, base + c])
            c0 = _sig(ch(_NCLS))
            c1 = _sig(ch(_NCLS + 5))
            sel = c0 >= c1
            conf = jnp.where(sel, c0, c1)
            bx = _sig(jnp.where(sel, ch(_NCLS + 1), ch(_NCLS + 6)))
            by = _sig(jnp.where(sel, ch(_NCLS + 2), ch(_NCLS + 7)))
            bw = _sig(jnp.where(sel, ch(_NCLS + 3), ch(_NCLS + 8)))
            bh = _sig(jnp.where(sel, ch(_NCLS + 4), ch(_NCLS + 9)))
            gx = (idxv % _GRID).astype(jnp.float32)
            gy = (idxv // _GRID).astype(jnp.float32)
            cx = (bx + gx) * _STRIDE
            cy = (by + gy) * _STRIDE
            w = (bw * float(_GRID)) * _STRIDE
            h = (bh * float(_GRID)) * _STRIDE
            mv = _sig(ch(0))
            cl = zeros
            for c in range(1, _NCLS):
                v = _sig(ch(c))
                cl = jnp.where(v > mv, float(c), cl)
                mv = jnp.maximum(mv, v)
            x1 = cx - w / 2.0
            y1 = cy - h / 2.0
            x2 = cx + w / 2.0
            y2 = cy + h / 2.0
            area = (x2 - x1) * (y2 - y1)
            x1r[sl] = x1
            y1r[sl] = y1
            x2r[sl] = x2
            y2r[sl] = y2
            arear[sl] = area
            cxr[sl] = cx
            cyr[sl] = cy
            wr[sl] = w
            hr[sl] = h
            confr[sl] = conf
            clsr[sl] = cl
            keepr[sl] = zeros
            x1v.append(x1)
            y1v.append(y1)
            x2v.append(x2)
            y2v.append(y2)
            areav.append(area)
            crv.append(jnp.where(inb, conf, 0.0))

        ones = jnp.ones((_L,), jnp.float32)
        lane0 = lane == 0

        def nms_step(i, cr):
            mxv = jnp.maximum(jnp.maximum(cr[0], cr[1]),
                              jnp.maximum(cr[2], cr[3]))
            mxv = _xlane(mxv, jnp.maximum, lane)
            active = mxv > _CONF_T
            candv = jnp.full((_L,), _NCELL - 1, jnp.int32)
            for j in range(_NCHUNK):
                idxv = lane + _L * j
                hit = jnp.where(cr[j] == mxv, idxv, _NCELL - 1)
                candv = jnp.minimum(candv, hit)
            candv = _xlane(candv, jnp.minimum, lane)
            plsc.store_scatter(keepr, [candv], ones, mask=lane0 & active)
            bx1 = plsc.load_gather(x1r, [candv])
            by1 = plsc.load_gather(y1r, [candv])
            bx2 = plsc.load_gather(x2r, [candv])
            by2 = plsc.load_gather(y2r, [candv])
            ba = plsc.load_gather(arear, [candv])
            out = []
            for j in range(_NCHUNK):
                idxv = lane + _L * j
                ix1 = jnp.maximum(x1v[j], bx1)
                iy1 = jnp.maximum(y1v[j], by1)
                ix2 = jnp.minimum(x2v[j], bx2)
                iy2 = jnp.minimum(y2v[j], by2)
                inter = jnp.maximum(ix2 - ix1, 0.0) * jnp.maximum(iy2 - iy1, 0.0)
                iou = inter / (areav[j] + ba - inter + 1e-9)
                gone = (iou > _IOU_T) | (idxv == candv)
                out.append(jnp.where(gone, 0.0, cr[j]))
            return tuple(out)

        lax.fori_loop(0, _NCELL, nms_step, tuple(crv))

        for j in range(_NCHUNK):
            sl = pl.ds(_L * j, _L)
            idxv = lane + _L * j
            inb = idxv < _NCELL
            kp = keepr[sl]
            for col, ref in enumerate((cxr, cyr, wr, hr, confr, clsr)):
                colv = jnp.full((_L,), col, jnp.int32)
                plsc.store_scatter(outv, [idxv, colv], ref[sl] * kp, mask=inb)

        pltpu.sync_copy(outv, out_hbm)


_vmem64 = lambda: pltpu.VMEM((_NCHUNK * _L,), jnp.float32)

_yolo_sc = functools.partial(
    pl.kernel,
    out_type=jax.ShapeDtypeStruct((_NCELL, 6), jnp.float32),
    mesh=plsc.VectorSubcoreMesh(core_axis_name="c", subcore_axis_name="s"),
    compiler_params=pltpu.CompilerParams(needs_layout_passes=False),
    scratch_types=[
        pltpu.VMEM((1, _NCELL * _NCH), jnp.float32),
        _vmem64(), _vmem64(), _vmem64(), _vmem64(), _vmem64(),
        _vmem64(), _vmem64(), _vmem64(), _vmem64(),
        _vmem64(), _vmem64(), _vmem64(),
        pltpu.VMEM((_NCELL, 6), jnp.float32),
    ],
)(_yolo_body)


@jax.jit
def kernel(x):
    return _yolo_sc(x)


# speedup vs baseline: 5.3527x; 1.0668x over previous
import functools

import jax
import jax.numpy as jnp
from jax import lax
from jax.experimental import pallas as pl
from jax.experimental.pallas import tpu as pltpu
from jax.experimental.pallas import tpu_sc as plsc

_GRID = 7
_NCELL = _GRID * _GRID
_NCH = 30
_NCLS = 20
_STRIDE = 64.0
_CONF_T = 0.5
_IOU_T = 0.5
_L = 16
_NCHUNK = 4

_GDN = lax.GatherDimensionNumbers(
    offset_dims=(), collapsed_slice_dims=(0,), start_index_map=(0,))


def _sig(v):
    return 1.0 / (1.0 + jnp.exp(-v))


def _perm(v, idx):
    return lax.gather(v, idx.reshape(_L, 1), _GDN, (1,),
                      mode=lax.GatherScatterMode.PROMISE_IN_BOUNDS)


def _xlane(v, op, lane):
    for s in (1, 2, 4, 8):
        v = op(v, _perm(v, lane ^ s))
    return v


def _yolo_body(x_hbm, out_hbm, xv, x1r, y1r, x2r, y2r, arear, cxr, cyr, wr,
               hr, confr, clsr, keepr, outv):
    @pl.when((lax.axis_index("c") == 0) & (lax.axis_index("s") == 0))
    def _():
        pltpu.sync_copy(x_hbm, xv)

        lane = lax.iota(jnp.int32, _L)
        zeros = jnp.zeros((_L,), jnp.float32)
        zeroi = jnp.zeros((_L,), jnp.int32)

        def chunk_gather(j, c):
            idxv = lane + _L * j
            base = idxv * _NCH
            if j == _NCHUNK - 1:
                return plsc.load_gather(xv, [zeroi, base + c],
                                        mask=idxv < _NCELL)
            return plsc.load_gather(xv, [zeroi, base + c])

        def cls_step(c, carry):
            mvs, cls_ = carry
            cf = c.astype(jnp.float32)
            out_mv, out_cl = [], []
            for j in range(_NCHUNK):
                v = _sig(chunk_gather(j, c))
                out_cl.append(jnp.where(v > mvs[j], cf, cls_[j]))
                out_mv.append(jnp.maximum(mvs[j], v))
            return (tuple(out_mv), tuple(out_cl))

        mv0 = tuple(_sig(chunk_gather(j, 0)) for j in range(_NCHUNK))
        _, clsv = lax.fori_loop(
            1, _NCLS, cls_step, (mv0, (zeros,) * _NCHUNK))

        x1v, y1v, x2v, y2v, areav, crv = [], [], [], [], [], []
        for j in range(_NCHUNK):
            sl = pl.ds(_L * j, _L)
            idxv = lane + _L * j
            inb = idxv < _NCELL
            ch = lambda c: chunk_gather(j, c)
            c0 = _sig(ch(_NCLS))
            c1 = _sig(ch(_NCLS + 5))
            sel = c0 >= c1
            conf = jnp.where(sel, c0, c1)
            bx = _sig(jnp.where(sel, ch(_NCLS + 1), ch(_NCLS + 6)))
            by = _sig(jnp.where(sel, ch(_NCLS + 2), ch(_NCLS + 7)))
            bw = _sig(jnp.where(sel, ch(_NCLS + 3), ch(_NCLS + 8)))
            bh = _sig(jnp.where(sel, ch(_NCLS + 4), ch(_NCLS + 9)))
            gx = (idxv % _GRID).astype(jnp.float32)
            gy = (idxv // _GRID).astype(jnp.float32)
            cx = (bx + gx) * _STRIDE
            cy = (by + gy) * _STRIDE
            w = (bw * float(_GRID)) * _STRIDE
            h = (bh * float(_GRID)) * _STRIDE
            cl = clsv[j]
            x1 = cx - w / 2.0
            y1 = cy - h / 2.0
            x2 = cx + w / 2.0
            y2 = cy + h / 2.0
            area = (x2 - x1) * (y2 - y1)
            x1r[sl] = x1
            y1r[sl] = y1
            x2r[sl] = x2
            y2r[sl] = y2
            arear[sl] = area
            cxr[sl] = cx
            cyr[sl] = cy
            wr[sl] = w
            hr[sl] = h
            confr[sl] = conf
            clsr[sl] = cl
            keepr[sl] = zeros
            x1v.append(x1)
            y1v.append(y1)
            x2v.append(x2)
            y2v.append(y2)
            areav.append(area)
            crv.append(jnp.where(inb, conf, 0.0))

        ones = jnp.ones((_L,), jnp.float32)
        lane0 = lane == 0

        def nms_step(i, cr):
            mxv = jnp.maximum(jnp.maximum(cr[0], cr[1]),
                              jnp.maximum(cr[2], cr[3]))
            mxv = _xlane(mxv, jnp.maximum, lane)
            active = mxv > _CONF_T
            candv = jnp.full((_L,), _NCELL - 1, jnp.int32)
            for j in range(_NCHUNK):
                idxv = lane + _L * j
                hit = jnp.where(cr[j] == mxv, idxv, _NCELL - 1)
                candv = jnp.minimum(candv, hit)
            candv = _xlane(candv, jnp.minimum, lane)
            plsc.store_scatter(keepr, [candv], ones, mask=lane0 & active)
            bx1 = plsc.load_gather(x1r, [candv])
            by1 = plsc.load_gather(y1r, [candv])
            bx2 = plsc.load_gather(x2r, [candv])
            by2 = plsc.load_gather(y2r, [candv])
            ba = plsc.load_gather(arear, [candv])
            out = []
            for j in range(_NCHUNK):
                idxv = lane + _L * j
                ix1 = jnp.maximum(x1v[j], bx1)
                iy1 = jnp.maximum(y1v[j], by1)
                ix2 = jnp.minimum(x2v[j], bx2)
                iy2 = jnp.minimum(y2v[j], by2)
                inter = jnp.maximum(ix2 - ix1, 0.0) * jnp.maximum(iy2 - iy1, 0.0)
                iou = inter / (areav[j] + ba - inter + 1e-9)
                gone = (iou > _IOU_T) | (idxv == candv)
                out.append(jnp.where(gone, 0.0, cr[j]))
            return tuple(out)

        lax.fori_loop(0, _NCELL, nms_step, tuple(crv))

        for j in range(_NCHUNK):
            sl = pl.ds(_L * j, _L)
            idxv = lane + _L * j
            inb = idxv < _NCELL
            kp = keepr[sl]
            for col, ref in enumerate((cxr, cyr, wr, hr, confr, clsr)):
                colv = jnp.full((_L,), col, jnp.int32)
                plsc.store_scatter(outv, [idxv, colv], ref[sl] * kp, mask=inb)

        pltpu.sync_copy(outv, out_hbm)


_vmem64 = lambda: pltpu.VMEM((_NCHUNK * _L,), jnp.float32)

_yolo_sc = functools.partial(
    pl.kernel,
    out_type=jax.ShapeDtypeStruct((_NCELL, 6), jnp.float32),
    mesh=plsc.VectorSubcoreMesh(core_axis_name="c", subcore_axis_name="s"),
    compiler_params=pltpu.CompilerParams(needs_layout_passes=False),
    scratch_types=[
        pltpu.VMEM((1, _NCELL * _NCH), jnp.float32),
        _vmem64(), _vmem64(), _vmem64(), _vmem64(), _vmem64(),
        _vmem64(), _vmem64(), _vmem64(), _vmem64(),
        _vmem64(), _vmem64(), _vmem64(),
        pltpu.VMEM((_NCELL, 6), jnp.float32),
    ],
)(_yolo_body)


@jax.jit
def kernel(x):
    return _yolo_sc(x)


# speedup vs baseline: 5.4624x; 1.0205x over previous
import functools

import jax
import jax.numpy as jnp
from jax import lax
from jax.experimental import pallas as pl
from jax.experimental.pallas import tpu as pltpu
from jax.experimental.pallas import tpu_sc as plsc

_GRID = 7
_NCELL = _GRID * _GRID
_NCH = 30
_NCLS = 20
_STRIDE = 64.0
_CONF_T = 0.5
_IOU_T = 0.5
_L = 16
_NCHUNK = 4

_GDN = lax.GatherDimensionNumbers(
    offset_dims=(), collapsed_slice_dims=(0,), start_index_map=(0,))


def _sig(v):
    return 1.0 / (1.0 + jnp.exp(-v))


def _perm(v, idx):
    return lax.gather(v, idx.reshape(_L, 1), _GDN, (1,),
                      mode=lax.GatherScatterMode.PROMISE_IN_BOUNDS)


def _xlane(v, op, lane):
    for s in (1, 2, 4, 8):
        v = op(v, _perm(v, lane ^ s))
    return v


def _yolo_body(x_hbm, out_hbm, xv, x1r, y1r, x2r, y2r, arear, cxr, cyr, wr,
               hr, confr, clsr, keepr, outv):
    @pl.when((lax.axis_index("c") == 0) & (lax.axis_index("s") == 0))
    def _():
        pltpu.sync_copy(x_hbm, xv)

        lane = lax.iota(jnp.int32, _L)
        zeros = jnp.zeros((_L,), jnp.float32)
        zeroi = jnp.zeros((_L,), jnp.int32)
        ones = jnp.ones((_L,), jnp.float32)
        lane0 = lane == 0

        def gather_ch(idxv, c):
            return plsc.load_gather(
                xv, [zeroi, idxv * _NCH + c], mask=idxv < _NCELL)

        def cls_step(c, carry):
            mvs, cls_ = carry
            cf = c.astype(jnp.float32)
            out_mv, out_cl = [], []
            for j in range(_NCHUNK):
                v = _sig(gather_ch(lane + _L * j, c))
                out_cl.append(jnp.where(v > mvs[j], cf, cls_[j]))
                out_mv.append(jnp.maximum(mvs[j], v))
            return (tuple(out_mv), tuple(out_cl))

        mv0 = tuple(_sig(gather_ch(lane + _L * j, 0)) for j in range(_NCHUNK))
        _, clsv = lax.fori_loop(1, _NCLS, cls_step, (mv0, (zeros,) * _NCHUNK))
        for j in range(_NCHUNK):
            clsr[pl.ds(_L * j, _L)] = clsv[j]

        def decode_step(j, carry):
            sl = pl.ds(j * _L, _L)
            idxv = lane + j * _L
            ch = lambda c: gather_ch(idxv, c)
            c0 = _sig(ch(_NCLS))
            c1 = _sig(ch(_NCLS + 5))
            sel = c0 >= c1
            conf = jnp.where(sel, c0, c1)
            bx = _sig(jnp.where(sel, ch(_NCLS + 1), ch(_NCLS + 6)))
            by = _sig(jnp.where(sel, ch(_NCLS + 2), ch(_NCLS + 7)))
            bw = _sig(jnp.where(sel, ch(_NCLS + 3), ch(_NCLS + 8)))
            bh = _sig(jnp.where(sel, ch(_NCLS + 4), ch(_NCLS + 9)))
            gx = (idxv % _GRID).astype(jnp.float32)
            gy = (idxv // _GRID).astype(jnp.float32)
            cx = (bx + gx) * _STRIDE
            cy = (by + gy) * _STRIDE
            w = (bw * float(_GRID)) * _STRIDE
            h = (bh * float(_GRID)) * _STRIDE
            x1 = cx - w / 2.0
            y1 = cy - h / 2.0
            x2 = cx + w / 2.0
            y2 = cy + h / 2.0
            x1r[sl] = x1
            y1r[sl] = y1
            x2r[sl] = x2
            y2r[sl] = y2
            arear[sl] = (x2 - x1) * (y2 - y1)
            cxr[sl] = cx
            cyr[sl] = cy
            wr[sl] = w
            hr[sl] = h
            confr[sl] = conf
            keepr[sl] = zeros
            return carry

        lax.fori_loop(0, _NCHUNK, decode_step, jnp.int32(0))

        x1v, y1v, x2v, y2v, areav, crv = [], [], [], [], [], []
        for j in range(_NCHUNK):
            sl = pl.ds(_L * j, _L)
            x1v.append(x1r[sl])
            y1v.append(y1r[sl])
            x2v.append(x2r[sl])
            y2v.append(y2r[sl])
            areav.append(arear[sl])
            crv.append(jnp.where(lane + _L * j < _NCELL, confr[sl], 0.0))

        def nms_step(i, cr):
            mxv = jnp.maximum(jnp.maximum(cr[0], cr[1]),
                              jnp.maximum(cr[2], cr[3]))
            mxv = _xlane(mxv, jnp.maximum, lane)
            active = mxv > _CONF_T
            candv = jnp.full((_L,), _NCELL - 1, jnp.int32)
            for j in range(_NCHUNK):
                hit = jnp.where(cr[j] == mxv, lane + _L * j, _NCELL - 1)
                candv = jnp.minimum(candv, hit)
            candv = _xlane(candv, jnp.minimum, lane)
            plsc.store_scatter(keepr, [candv], ones, mask=lane0 & active)
            bx1 = plsc.load_gather(x1r, [candv])
            by1 = plsc.load_gather(y1r, [candv])
            bx2 = plsc.load_gather(x2r, [candv])
            by2 = plsc.load_gather(y2r, [candv])
            ba = plsc.load_gather(arear, [candv])
            out = []
            for j in range(_NCHUNK):
                ix1 = jnp.maximum(x1v[j], bx1)
                iy1 = jnp.maximum(y1v[j], by1)
                ix2 = jnp.minimum(x2v[j], bx2)
                iy2 = jnp.minimum(y2v[j], by2)
                inter = jnp.maximum(ix2 - ix1, 0.0) * jnp.maximum(iy2 - iy1, 0.0)
                iou = inter / (areav[j] + ba - inter + 1e-9)
                gone = (iou > _IOU_T) | (lane + _L * j == candv)
                out.append(jnp.where(gone, 0.0, cr[j]))
            return tuple(out)

        lax.fori_loop(0, _NCELL, nms_step, tuple(crv))

        def out_step(j, carry):
            sl = pl.ds(j * _L, _L)
            idxv = lane + j * _L
            inb = idxv < _NCELL
            kp = keepr[sl]
            for col, ref in enumerate((cxr, cyr, wr, hr, confr, clsr)):
                colv = jnp.full((_L,), col, jnp.int32)
                plsc.store_scatter(outv, [idxv, colv], ref[sl] * kp, mask=inb)
            return carry

        lax.fori_loop(0, _NCHUNK, out_step, jnp.int32(0))

        pltpu.sync_copy(outv, out_hbm)


_vmem64 = lambda: pltpu.VMEM((_NCHUNK * _L,), jnp.float32)

_yolo_sc = functools.partial(
    pl.kernel,
    out_type=jax.ShapeDtypeStruct((_NCELL, 6), jnp.float32),
    mesh=plsc.VectorSubcoreMesh(core_axis_name="c", subcore_axis_name="s"),
    compiler_params=pltpu.CompilerParams(needs_layout_passes=False),
    scratch_types=[
        pltpu.VMEM((1, _NCELL * _NCH), jnp.float32),
        _vmem64(), _vmem64(), _vmem64(), _vmem64(), _vmem64(),
        _vmem64(), _vmem64(), _vmem64(), _vmem64(),
        _vmem64(), _vmem64(), _vmem64(),
        pltpu.VMEM((_NCELL, 6), jnp.float32),
    ],
)(_yolo_body)


@jax.jit
def kernel(x):
    return _yolo_sc(x)


# speedup vs baseline: 5.4999x; 1.0069x over previous
import functools

import jax
import jax.numpy as jnp
from jax import lax
from jax.experimental import pallas as pl
from jax.experimental.pallas import tpu as pltpu
from jax.experimental.pallas import tpu_sc as plsc

_GRID = 7
_NCELL = _GRID * _GRID
_NCH = 30
_NCLS = 20
_STRIDE = 64.0
_CONF_T = 0.5
_IOU_T = 0.5
_L = 16
_NCHUNK = 4

_GDN = lax.GatherDimensionNumbers(
    offset_dims=(), collapsed_slice_dims=(0,), start_index_map=(0,))


def _sig(v):
    return 1.0 / (1.0 + jnp.exp(-v))


def _perm(v, idx):
    return lax.gather(v, idx.reshape(_L, 1), _GDN, (1,),
                      mode=lax.GatherScatterMode.PROMISE_IN_BOUNDS)


def _xlane(v, op, lane):
    for s in (1, 2, 4, 8):
        v = op(v, _perm(v, lane ^ s))
    return v


def _yolo_body(x_hbm, out_hbm, xv, x1r, y1r, x2r, y2r, arear, cxr, cyr, wr,
               hr, confr, clsr, keepr, outv):
    @pl.when((lax.axis_index("c") == 0) & (lax.axis_index("s") == 0))
    def _():
        pltpu.sync_copy(x_hbm, xv)

        lane = lax.iota(jnp.int32, _L)
        zeros = jnp.zeros((_L,), jnp.float32)
        zeroi = jnp.zeros((_L,), jnp.int32)
        ones = jnp.ones((_L,), jnp.float32)
        lane0 = lane == 0

        def gather_ch(idxv, c):
            return plsc.load_gather(
                xv, [zeroi, idxv * _NCH + c], mask=idxv < _NCELL)

        def cls_step(c, carry):
            mvs, cls_ = carry
            cf = c.astype(jnp.float32)
            out_mv, out_cl = [], []
            for j in range(_NCHUNK):
                v = _sig(gather_ch(lane + _L * j, c))
                out_cl.append(jnp.where(v > mvs[j], cf, cls_[j]))
                out_mv.append(jnp.maximum(mvs[j], v))
            return (tuple(out_mv), tuple(out_cl))

        mv0 = tuple(_sig(gather_ch(lane + _L * j, 0)) for j in range(_NCHUNK))
        _, clsv = lax.fori_loop(1, _NCLS, cls_step, (mv0, (zeros,) * _NCHUNK))
        for j in range(_NCHUNK):
            clsr[pl.ds(_L * j, _L)] = clsv[j]

        def decode_step(j, carry):
            sl = pl.ds(j * _L, _L)
            idxv = lane + j * _L
            ch = lambda c: gather_ch(idxv, c)
            c0 = _sig(ch(_NCLS))
            c1 = _sig(ch(_NCLS + 5))
            sel = c0 >= c1
            conf = jnp.where(sel, c0, c1)
            bx = _sig(jnp.where(sel, ch(_NCLS + 1), ch(_NCLS + 6)))
            by = _sig(jnp.where(sel, ch(_NCLS + 2), ch(_NCLS + 7)))
            bw = _sig(jnp.where(sel, ch(_NCLS + 3), ch(_NCLS + 8)))
            bh = _sig(jnp.where(sel, ch(_NCLS + 4), ch(_NCLS + 9)))
            gx = (idxv % _GRID).astype(jnp.float32)
            gy = (idxv // _GRID).astype(jnp.float32)
            cx = (bx + gx) * _STRIDE
            cy = (by + gy) * _STRIDE
            w = (bw * float(_GRID)) * _STRIDE
            h = (bh * float(_GRID)) * _STRIDE
            x1 = cx - w / 2.0
            y1 = cy - h / 2.0
            x2 = cx + w / 2.0
            y2 = cy + h / 2.0
            x1r[sl] = x1
            y1r[sl] = y1
            x2r[sl] = x2
            y2r[sl] = y2
            arear[sl] = (x2 - x1) * (y2 - y1)
            cxr[sl] = cx
            cyr[sl] = cy
            wr[sl] = w
            hr[sl] = h
            confr[sl] = conf
            keepr[sl] = zeros
            return carry

        lax.fori_loop(0, _NCHUNK, decode_step, jnp.int32(0))

        x1v, y1v, x2v, y2v, areav, crv = [], [], [], [], [], []
        for j in range(_NCHUNK):
            sl = pl.ds(_L * j, _L)
            x1v.append(x1r[sl])
            y1v.append(y1r[sl])
            x2v.append(x2r[sl])
            y2v.append(y2r[sl])
            areav.append(arear[sl])
            crv.append(jnp.where(lane + _L * j < _NCELL, confr[sl], 0.0))

        def _maxv(cr):
            mxv = jnp.maximum(jnp.maximum(cr[0], cr[1]),
                              jnp.maximum(cr[2], cr[3]))
            return _xlane(mxv, jnp.maximum, lane)

        def nms_cond(carry):
            return carry[0][0] > _CONF_T

        def nms_step(carry):
            mxv, cr = carry[0], carry[1:]
            candv = jnp.full((_L,), _NCELL - 1, jnp.int32)
            for j in range(_NCHUNK):
                hit = jnp.where(cr[j] == mxv, lane + _L * j, _NCELL - 1)
                candv = jnp.minimum(candv, hit)
            candv = _xlane(candv, jnp.minimum, lane)
            plsc.store_scatter(keepr, [candv], ones, mask=lane0)
            bx1 = plsc.load_gather(x1r, [candv])
            by1 = plsc.load_gather(y1r, [candv])
            bx2 = plsc.load_gather(x2r, [candv])
            by2 = plsc.load_gather(y2r, [candv])
            ba = plsc.load_gather(arear, [candv])
            out = []
            for j in range(_NCHUNK):
                ix1 = jnp.maximum(x1v[j], bx1)
                iy1 = jnp.maximum(y1v[j], by1)
                ix2 = jnp.minimum(x2v[j], bx2)
                iy2 = jnp.minimum(y2v[j], by2)
                inter = jnp.maximum(ix2 - ix1, 0.0) * jnp.maximum(iy2 - iy1, 0.0)
                iou = inter / (areav[j] + ba - inter + 1e-9)
                gone = (iou > _IOU_T) | (lane + _L * j == candv)
                out.append(jnp.where(gone, 0.0, cr[j]))
            return (_maxv(out), *out)

        lax.while_loop(nms_cond, nms_step, (_maxv(crv), *crv))

        def out_step(j, carry):
            sl = pl.ds(j * _L, _L)
            idxv = lane + j * _L
            inb = idxv < _NCELL
            kp = keepr[sl]
            for col, ref in enumerate((cxr, cyr, wr, hr, confr, clsr)):
                colv = jnp.full((_L,), col, jnp.int32)
                plsc.store_scatter(outv, [idxv, colv], ref[sl] * kp, mask=inb)
            return carry

        lax.fori_loop(0, _NCHUNK, out_step, jnp.int32(0))

        pltpu.sync_copy(outv, out_hbm)


_vmem64 = lambda: pltpu.VMEM((_NCHUNK * _L,), jnp.float32)

_yolo_sc = functools.partial(
    pl.kernel,
    out_type=jax.ShapeDtypeStruct((_NCELL, 6), jnp.float32),
    mesh=plsc.VectorSubcoreMesh(core_axis_name="c", subcore_axis_name="s"),
    compiler_params=pltpu.CompilerParams(needs_layout_passes=False),
    scratch_types=[
        pltpu.VMEM((1, _NCELL * _NCH), jnp.float32),
        _vmem64(), _vmem64(), _vmem64(), _vmem64(), _vmem64(),
        _vmem64(), _vmem64(), _vmem64(), _vmem64(),
        _vmem64(), _vmem64(), _vmem64(),
        pltpu.VMEM((_NCELL, 6), jnp.float32),
    ],
)(_yolo_body)


@jax.jit
def kernel(x):
    return _yolo_sc(x)


# speedup vs baseline: 5.8420x; 1.0622x over previous
import functools

import jax
import jax.numpy as jnp
from jax import lax
from jax.experimental import pallas as pl
from jax.experimental.pallas import tpu as pltpu
from jax.experimental.pallas import tpu_sc as plsc

_GRID = 7
_NCELL = _GRID * _GRID
_NCH = 30
_NCLS = 20
_STRIDE = 64.0
_CONF_T = 0.5
_IOU_T = 0.5
_L = 16
_NCHUNK = 4

_GDN = lax.GatherDimensionNumbers(
    offset_dims=(), collapsed_slice_dims=(0,), start_index_map=(0,))


def _sig(v):
    return 1.0 / (1.0 + jnp.exp(-v))


def _perm(v, idx):
    return lax.gather(v, idx.reshape(_L, 1), _GDN, (1,),
                      mode=lax.GatherScatterMode.PROMISE_IN_BOUNDS)


def _xlane(v, op, lane):
    for s in (1, 2, 4, 8):
        v = op(v, _perm(v, lane ^ s))
    return v


def _yolo_body(x_hbm, out_hbm, xv, x1r, y1r, x2r, y2r, arear, cxr, cyr, wr,
               hr, confr, clsr, keepr, outv):
    @pl.when((lax.axis_index("c") == 0) & (lax.axis_index("s") == 0))
    def _():
        pltpu.sync_copy(x_hbm, xv)

        lane = lax.iota(jnp.int32, _L)
        zeros = jnp.zeros((_L,), jnp.float32)
        zeroi = jnp.zeros((_L,), jnp.int32)
        ones = jnp.ones((_L,), jnp.float32)
        lane0 = lane == 0

        def gather_ch(idxv, c):
            return plsc.load_gather(
                xv, [zeroi, idxv * _NCH + c], mask=idxv < _NCELL)

        def cls_step(c, carry):
            mvs, cls_ = carry
            cf = c.astype(jnp.float32)
            out_mv, out_cl = [], []
            for j in range(_NCHUNK):
                v = _sig(gather_ch(lane + _L * j, c))
                out_cl.append(jnp.where(v > mvs[j], cf, cls_[j]))
                out_mv.append(jnp.maximum(mvs[j], v))
            return (tuple(out_mv), tuple(out_cl))

        mv0 = tuple(_sig(gather_ch(lane + _L * j, 0)) for j in range(_NCHUNK))
        _, clsv = lax.fori_loop(1, _NCLS, cls_step, (mv0, (zeros,) * _NCHUNK))
        for j in range(_NCHUNK):
            clsr[pl.ds(_L * j, _L)] = clsv[j]

        def decode_step(j, carry):
            sl = pl.ds(j * _L, _L)
            idxv = lane + j * _L
            ch = lambda c: gather_ch(idxv, c)
            c0 = _sig(ch(_NCLS))
            c1 = _sig(ch(_NCLS + 5))
            sel = c0 >= c1
            conf = jnp.where(sel, c0, c1)
            bx = _sig(jnp.where(sel, ch(_NCLS + 1), ch(_NCLS + 6)))
            by = _sig(jnp.where(sel, ch(_NCLS + 2), ch(_NCLS + 7)))
            bw = _sig(jnp.where(sel, ch(_NCLS + 3), ch(_NCLS + 8)))
            bh = _sig(jnp.where(sel, ch(_NCLS + 4), ch(_NCLS + 9)))
            gx = (idxv % _GRID).astype(jnp.float32)
            gy = (idxv // _GRID).astype(jnp.float32)
            cx = (bx + gx) * _STRIDE
            cy = (by + gy) * _STRIDE
            w = (bw * float(_GRID)) * _STRIDE
            h = (bh * float(_GRID)) * _STRIDE
            x1 = cx - w / 2.0
            y1 = cy - h / 2.0
            x2 = cx + w / 2.0
            y2 = cy + h / 2.0
            x1r[sl] = x1
            y1r[sl] = y1
            x2r[sl] = x2
            y2r[sl] = y2
            arear[sl] = (x2 - x1) * (y2 - y1)
            cxr[sl] = cx
            cyr[sl] = cy
            wr[sl] = w
            hr[sl] = h
            confr[sl] = conf
            keepr[sl] = zeros
            return carry

        lax.fori_loop(0, _NCHUNK, decode_step, jnp.int32(0))

        x1v, y1v, x2v, y2v, areav, crv = [], [], [], [], [], []
        for j in range(_NCHUNK):
            sl = pl.ds(_L * j, _L)
            x1v.append(x1r[sl])
            y1v.append(y1r[sl])
            x2v.append(x2r[sl])
            y2v.append(y2r[sl])
            areav.append(arear[sl])
            crv.append(jnp.where(lane + _L * j < _NCELL, confr[sl], 0.0))

        def _maxv(cr):
            mxv = jnp.maximum(jnp.maximum(cr[0], cr[1]),
                              jnp.maximum(cr[2], cr[3]))
            return _xlane(mxv, jnp.maximum, lane)

        def nms_cond(carry):
            return carry[0][0] > _CONF_T

        def nms_step(carry):
            mxv, cr = carry[0], carry[1:]
            candv = jnp.full((_L,), _NCELL - 1, jnp.int32)
            for j in range(_NCHUNK):
                hit = jnp.where(cr[j] == mxv, lane + _L * j, _NCELL - 1)
                candv = jnp.minimum(candv, hit)
            candv = _xlane(candv, jnp.minimum, lane)
            plsc.store_scatter(keepr, [candv], ones, mask=lane0)
            bx1 = plsc.load_gather(x1r, [candv])
            by1 = plsc.load_gather(y1r, [candv])
            bx2 = plsc.load_gather(x2r, [candv])
            by2 = plsc.load_gather(y2r, [candv])
            ba = plsc.load_gather(arear, [candv])
            out = []
            for j in range(_NCHUNK):
                ix1 = jnp.maximum(x1v[j], bx1)
                iy1 = jnp.maximum(y1v[j], by1)
                ix2 = jnp.minimum(x2v[j], bx2)
                iy2 = jnp.minimum(y2v[j], by2)
                inter = jnp.maximum(ix2 - ix1, 0.0) * jnp.maximum(iy2 - iy1, 0.0)
                iou = inter / (areav[j] + ba - inter + 1e-9)
                gone = (iou > _IOU_T) | (lane + _L * j == candv)
                out.append(jnp.where(gone, 0.0, cr[j]))
            return (_maxv(out), *out)

        lax.while_loop(nms_cond, nms_step, (_maxv(crv), *crv))

        def out_step(j, carry):
            sl = pl.ds(j * _L, _L)
            idxv = lane + j * _L
            inb = idxv < _NCELL
            kp = keepr[sl]
            for col, ref in enumerate((cxr, cyr, wr, hr, confr, clsr)):
                colv = jnp.full((_L,), col, jnp.int32)
                plsc.store_scatter(outv, [idxv, colv], ref[sl] * kp, mask=inb)
            return carry

        lax.fori_loop(0, _NCHUNK, out_step, jnp.int32(0))

        pltpu.sync_copy(outv, out_hbm)


_vmem64 = lambda: pltpu.VMEM((_NCHUNK * _L,), jnp.float32)

_yolo_sc = functools.partial(
    pl.kernel,
    out_type=jax.ShapeDtypeStruct((_NCELL, 6), jnp.float32),
    mesh=plsc.VectorSubcoreMesh(core_axis_name="c", subcore_axis_name="s",
                                num_cores=1),
    compiler_params=pltpu.CompilerParams(needs_layout_passes=False,
                                         skip_device_barrier=True),
    scratch_types=[
        pltpu.VMEM((1, _NCELL * _NCH), jnp.float32),
        _vmem64(), _vmem64(), _vmem64(), _vmem64(), _vmem64(),
        _vmem64(), _vmem64(), _vmem64(), _vmem64(),
        _vmem64(), _vmem64(), _vmem64(),
        pltpu.VMEM((_NCELL, 6), jnp.float32),
    ],
)(_yolo_body)


@jax.jit
def kernel(x):
    return _yolo_sc(x)
